# baseline (speedup 1.0000x reference)
import functools

import jax
import jax.numpy as jnp
from jax import lax
from jax.experimental import pallas as pl
from jax.experimental.pallas import tpu as pltpu
from jax.experimental.pallas import tpu_sc as plsc

N = 10000
E = 160000
H = 512
OUT = 250
L = 4


R = 1024
G = (N + R - 1) // R


def _leaky(x):
    return jnp.where(x >= 0, x, 0.2 * x)


def _dotT(a, w):
    return jax.lax.dot_general(a, w, (((1,), (1,)), ((), ())),
                               preferred_element_type=jnp.float32)


def _pre_body(xw, wpre, wpost, o):
    h = _leaky(_dotT(xw[...], wpre[...]))
    o[...] = _leaky(_dotT(h, wpost[...]))


def _pre(xw, wpre, wpost):
    return pl.pallas_call(
        _pre_body,
        grid=(G,),
        in_specs=[
            pl.BlockSpec((R, H), lambda i: (i, 0)),
            pl.BlockSpec((H, H), lambda i: (0, 0)),
            pl.BlockSpec((H, H), lambda i: (0, 0)),
        ],
        out_specs=pl.BlockSpec((R, H), lambda i: (i, 0)),
        out_shape=jax.ShapeDtypeStruct((N, H), jnp.float32),
    )(xw, wpre, wpost)


def _cat(sq):
    a = sq[...]
    return jnp.concatenate([a[0], a[1], a[2], a[3]], axis=-1)


def _self_body(x, wrs, bsum, o):
    o[...] = _dotT(x[...], wrs[...]) + bsum[...]


def _self(x, wrs, bsum):
    blk = pl.BlockSpec((R, H), lambda i: (i, 0))
    return pl.pallas_call(
        _self_body,
        grid=(G,),
        in_specs=[blk, pl.BlockSpec((H, H), lambda i: (0, 0)),
                  pl.BlockSpec((1, H), lambda i: (0, 0))],
        out_specs=blk,
        out_shape=jax.ShapeDtypeStruct((N, H), jnp.float32),
    )(x, wrs, bsum)


def _acc_body(t, s, cnt, wl, o):
    ic = 1.0 / jnp.maximum(cnt[...], 1.0)
    o[...] = t[...] + _dotT(_cat(s) * ic, wl[...])


def _acc_final_body(t, s, cnt, wl, o):
    ic = 1.0 / jnp.maximum(cnt[...], 1.0)
    o[...] = _leaky((t[...] + _dotT(_cat(s) * ic, wl[...])) * (1.0 / 3.0))


def _acc(t, s, cnt, wl, final):
    blk = pl.BlockSpec((R, H), lambda i: (i, 0))
    sblk = pl.BlockSpec((4, R, QW), lambda i: (0, i, 0))
    return pl.pallas_call(
        _acc_final_body if final else _acc_body,
        grid=(G,),
        in_specs=[blk, sblk, pl.BlockSpec((R, 1), lambda i: (i, 0)),
                  pl.BlockSpec((H, H), lambda i: (0, 0))],
        out_specs=blk,
        out_shape=jax.ShapeDtypeStruct((N, H), jnp.float32),
    )(t, s, cnt, wl)


def _final_body(x, wlin, blin, o):
    o[...] = _dotT(x[...], wlin[...]) + blin[...]


def _final(x, wlin, blin):
    return pl.pallas_call(
        _final_body,
        grid=(G,),
        in_specs=[
            pl.BlockSpec((R, H), lambda i: (i, 0)),
            pl.BlockSpec((OUT, H), lambda i: (0, 0)),
            pl.BlockSpec((1, OUT), lambda i: (0, 0)),
        ],
        out_specs=pl.BlockSpec((R, OUT), lambda i: (i, 0)),
        out_shape=jax.ShapeDtypeStruct((N, OUT), jnp.float32),
    )(x, wlin, blin)


NC, NS = 2, 16
B = 128
NBAT = 80
HALF = NBAT // 2
EPT = NBAT * B
EPAD = EPT * NS
NPAD = 10240
ZPT = NPAD // NS
FPT = N // NS
QW = 128

_MESH = plsc.VectorSubcoreMesh(core_axis_name="c", subcore_axis_name="s")


CPT = NPAD // 2 // NS


def _cnt_body(d0, d1, d2, z1, ones1, o0, o1, o2, acc, ones_v, dst_v, zb, fb):
    c = lax.axis_index("c")
    s = lax.axis_index("s")
    pltpu.sync_copy(ones1, ones_v)
    pltpu.sync_copy(z1, zb)
    for rel in range(3):
        dst_hbm = (d0, d1, d2)[rel]
        out = (o0, o1, o2)[rel]
        pltpu.sync_copy(zb, acc.at[pl.ds(s * ZPT, ZPT)])
        pltpu.sync_copy(dst_hbm.at[s], dst_v)
        plsc.subcore_barrier()

        @pl.loop(0, NBAT)
        def _(j):
            pltpu.sync_copy(ones_v, acc.at[dst_v.at[j]], add=True)

        plsc.subcore_barrier()
        off = c * (NPAD // 2) + s * CPT
        pltpu.sync_copy(acc.at[pl.ds(off, CPT)], fb)
        pltpu.sync_copy(fb, out.at[pl.ds(off, CPT)])
        plsc.subcore_barrier()


@functools.partial(
    pl.kernel, mesh=_MESH,
    out_type=[jax.ShapeDtypeStruct((NPAD,), jnp.float32)] * 3,
    scratch_types=[
        pltpu.VMEM_SHARED((NPAD,), jnp.float32),
        pltpu.VMEM((B,), jnp.float32),
        pltpu.VMEM((NBAT, B), jnp.int32),
        pltpu.VMEM((ZPT,), jnp.float32),
        pltpu.VMEM((CPT,), jnp.float32),
    ],
)
def _counts(*args):
    _cnt_body(*args)


def _agg_body(x2d, src4, dst_hbm, z2, out, acc, src_v, dst_v,
              rows0, rows1, gsem0, gsem1, ssem0, ssem1):
    c = lax.axis_index("c")
    s = lax.axis_index("s")
    if True:
        for qi in range(2):
            q = c * 2 + qi
            pltpu.sync_copy(z2, rows0)
            for k in range(ZPT // B):
                pltpu.sync_copy(rows0, acc.at[pl.ds(s * ZPT + k * B, B)])
            plsc.subcore_barrier()

            for h in range(2):
                pltpu.sync_copy(src4.at[q, s, pl.ds(h * HALF, HALF)], src_v)
                pltpu.sync_copy(dst_hbm.at[s, pl.ds(h * HALF, HALF)], dst_v)

                def gth(j, rw, sem):
                    pltpu.async_copy(x2d.at[src_v.at[j]], rw, sem)

                def gwait(j, rw, sem):
                    pltpu.make_async_copy(x2d.at[src_v.at[j]], rw, sem).wait()

                def sct(j, rw, sem):
                    pltpu.async_copy(rw, acc.at[dst_v.at[j]], sem, add=True)

                def swait(rw, sem):
                    pltpu.make_async_copy(rw, acc.at[dst_v.at[0]], sem).wait()

                gth(0, rows0, gsem0)
                gwait(0, rows0, gsem0)
                sct(0, rows0, ssem0)
                gth(1, rows1, gsem1)

                @pl.loop(1, HALF - 1, step=2)
                def _(j):
                    gwait(j, rows1, gsem1)
                    sct(j, rows1, ssem1)
                    swait(rows0, ssem0)
                    gth(j + 1, rows0, gsem0)
                    gwait(j + 1, rows0, gsem0)
                    sct(j + 1, rows0, ssem0)
                    swait(rows1, ssem1)
                    gth(j + 2, rows1, gsem1)

                gwait(HALF - 1, rows1, gsem1)
                sct(HALF - 1, rows1, ssem1)
                swait(rows0, ssem0)
                swait(rows1, ssem1)

            plsc.subcore_barrier()
            for k in range(ZPT // B):
                r0 = s * ZPT + k * B
                pltpu.sync_copy(acc.at[pl.ds(r0, B)], rows0)
                pltpu.sync_copy(rows0, out.at[q, pl.ds(r0, B)])
            plsc.subcore_barrier()


@functools.partial(
    pl.kernel, mesh=_MESH,
    out_type=jax.ShapeDtypeStruct((4, NPAD, QW), jnp.float32),
    scratch_types=[
        pltpu.VMEM_SHARED((NPAD, QW), jnp.float32),
        pltpu.VMEM((HALF, B), jnp.int32),
        pltpu.VMEM((HALF, B), jnp.int32),
        pltpu.VMEM((B, QW), jnp.float32),
        pltpu.VMEM((B, QW), jnp.float32),
        pltpu.SemaphoreType.DMA,
        pltpu.SemaphoreType.DMA,
        pltpu.SemaphoreType.DMA,
        pltpu.SemaphoreType.DMA,
    ],
)
def _aggregate(*args):
    _agg_body(*args)


def kernel(x_window, edge_index_near, edge_index_close, edge_index_sim,
           W_pre, W_post,
           Wl_0_near, bl_0_near, Wr_0_near,
           Wl_0_close, bl_0_close, Wr_0_close,
           Wl_0_sim, bl_0_sim, Wr_0_sim,
           Wl_1_near, bl_1_near, Wr_1_near,
           Wl_1_close, bl_1_close, Wr_1_close,
           Wl_1_sim, bl_1_sim, Wr_1_sim,
           Wl_2_near, bl_2_near, Wr_2_near,
           Wl_2_close, bl_2_close, Wr_2_close,
           Wl_2_sim, bl_2_sim, Wr_2_sim,
           Wl_3_near, bl_3_near, Wr_3_near,
           Wl_3_close, bl_3_close, Wr_3_close,
           Wl_3_sim, bl_3_sim, Wr_3_sim,
           W_lin, b_lin):
    params = dict(locals())
    rels = ("near", "close", "sim")

    pad = jnp.arange(EPAD - E, dtype=jnp.int32)
    pad_src = (pad * 41) % N
    pad_dst = N + (pad % (NPAD - N))
    src4s, dst3s = [], []
    for r in rels:
        ei = params["edge_index_" + r]
        src = jnp.concatenate([ei[0], pad_src])
        dst = jnp.concatenate([ei[1], pad_dst])
        src4 = (4 * src)[None, :] + jnp.arange(4, dtype=jnp.int32)[:, None]
        src4s.append(src4.reshape(4, NS, NBAT, B))
        dst3s.append(dst.reshape(NS, NBAT, B))

    zeros1 = jnp.zeros((ZPT,), jnp.float32)
    ones1 = jnp.ones((B,), jnp.float32)
    zeros2 = jnp.zeros((B, QW), jnp.float32)

    cnts = _counts(dst3s[0], dst3s[1], dst3s[2], zeros1, ones1)
    cnts = [cc[:N, None] for cc in cnts]

    x = _pre(x_window, W_pre, W_post)

    for l in range(L):
        wrs = (params["Wr_%d_near" % l] + params["Wr_%d_close" % l]
               + params["Wr_%d_sim" % l])
        bsum = (params["bl_%d_near" % l] + params["bl_%d_close" % l]
                + params["bl_%d_sim" % l])[None, :]
        x2d = x.reshape(4 * N, QW)
        t = _self(x, wrs, bsum)
        for i, r in enumerate(rels):
            sr = _aggregate(x2d, src4s[i], dst3s[i], zeros2)
            t = _acc(t, sr, cnts[i], params["Wl_%d_%s" % (l, r)],
                     final=(i == 2))
        x = t

    return _final(x, W_lin, b_lin[None, :])


# speedup vs baseline: 1.1671x; 1.1671x over previous
import functools

import jax
import jax.numpy as jnp
from jax import lax
from jax.experimental import pallas as pl
from jax.experimental.pallas import tpu as pltpu
from jax.experimental.pallas import tpu_sc as plsc

N = 10000
E = 160000
H = 512
OUT = 250
L = 4


R = 1024
G = (N + R - 1) // R


def _leaky(x):
    return jnp.where(x >= 0, x, 0.2 * x)


def _dotT(a, w):
    return jax.lax.dot_general(a, w, (((1,), (1,)), ((), ())),
                               preferred_element_type=jnp.float32)


def _pre_body(xw, wpre, wpost, o):
    h = _leaky(_dotT(xw[...], wpre[...]))
    o[...] = _leaky(_dotT(h, wpost[...]))


def _pre(xw, wpre, wpost):
    return pl.pallas_call(
        _pre_body,
        grid=(G,),
        in_specs=[
            pl.BlockSpec((R, H), lambda i: (i, 0)),
            pl.BlockSpec((H, H), lambda i: (0, 0)),
            pl.BlockSpec((H, H), lambda i: (0, 0)),
        ],
        out_specs=pl.BlockSpec((R, H), lambda i: (i, 0)),
        out_shape=jax.ShapeDtypeStruct((N, H), jnp.float32),
    )(xw, wpre, wpost)


def _cat(sq):
    a = sq[...]
    return jnp.concatenate([a[0], a[1], a[2], a[3]], axis=-1)


def _self_body(x, wrs, bsum, o):
    o[...] = _dotT(x[...], wrs[...]) + bsum[...]


def _self(x, wrs, bsum):
    blk = pl.BlockSpec((R, H), lambda i: (i, 0))
    return pl.pallas_call(
        _self_body,
        grid=(G,),
        in_specs=[blk, pl.BlockSpec((H, H), lambda i: (0, 0)),
                  pl.BlockSpec((1, H), lambda i: (0, 0))],
        out_specs=blk,
        out_shape=jax.ShapeDtypeStruct((N, H), jnp.float32),
    )(x, wrs, bsum)


def _acc_body(t, s, cnt, wl, o):
    ic = 1.0 / jnp.maximum(cnt[...], 1.0)
    o[...] = t[...] + _dotT(_cat(s) * ic, wl[...])


def _acc_final_body(t, s, cnt, wl, o):
    ic = 1.0 / jnp.maximum(cnt[...], 1.0)
    o[...] = _leaky((t[...] + _dotT(_cat(s) * ic, wl[...])) * (1.0 / 3.0))


def _acc(t, s, cnt, wl, final):
    blk = pl.BlockSpec((R, H), lambda i: (i, 0))
    sblk = pl.BlockSpec((4, R, QW), lambda i: (0, i, 0))
    return pl.pallas_call(
        _acc_final_body if final else _acc_body,
        grid=(G,),
        in_specs=[blk, sblk, pl.BlockSpec((R, 1), lambda i: (i, 0)),
                  pl.BlockSpec((H, H), lambda i: (0, 0))],
        out_specs=blk,
        out_shape=jax.ShapeDtypeStruct((N, H), jnp.float32),
    )(t, s, cnt, wl)


def _final_body(x, wlin, blin, o):
    o[...] = _dotT(x[...], wlin[...]) + blin[...]


def _final(x, wlin, blin):
    return pl.pallas_call(
        _final_body,
        grid=(G,),
        in_specs=[
            pl.BlockSpec((R, H), lambda i: (i, 0)),
            pl.BlockSpec((OUT, H), lambda i: (0, 0)),
            pl.BlockSpec((1, OUT), lambda i: (0, 0)),
        ],
        out_specs=pl.BlockSpec((R, OUT), lambda i: (i, 0)),
        out_shape=jax.ShapeDtypeStruct((N, OUT), jnp.float32),
    )(x, wlin, blin)


NC, NS = 2, 16
B = 128
NBAT = 80
HALF = NBAT // 2
EPT = NBAT * B
EPAD = EPT * NS
NPAD = 10240
ZPT = NPAD // NS
FPT = N // NS
QW = 128

_MESH = plsc.VectorSubcoreMesh(core_axis_name="c", subcore_axis_name="s")


CPT = NPAD // 2 // NS


def _cnt_body(d0, d1, d2, z1, ones1, o0, o1, o2, acc, ones_v, dst_v, zb, fb):
    c = lax.axis_index("c")
    s = lax.axis_index("s")
    pltpu.sync_copy(ones1, ones_v)
    pltpu.sync_copy(z1, zb)
    for rel in range(3):
        dst_hbm = (d0, d1, d2)[rel]
        out = (o0, o1, o2)[rel]
        pltpu.sync_copy(zb, acc.at[pl.ds(s * ZPT, ZPT)])
        pltpu.sync_copy(dst_hbm.at[s], dst_v)
        plsc.subcore_barrier()

        @pl.loop(0, NBAT)
        def _(j):
            pltpu.sync_copy(ones_v, acc.at[dst_v.at[j]], add=True)

        plsc.subcore_barrier()
        off = c * (NPAD // 2) + s * CPT
        pltpu.sync_copy(acc.at[pl.ds(off, CPT)], fb)
        pltpu.sync_copy(fb, out.at[pl.ds(off, CPT)])
        plsc.subcore_barrier()


@functools.partial(
    pl.kernel, mesh=_MESH,
    out_type=[jax.ShapeDtypeStruct((NPAD,), jnp.float32)] * 3,
    scratch_types=[
        pltpu.VMEM_SHARED((NPAD,), jnp.float32),
        pltpu.VMEM((B,), jnp.float32),
        pltpu.VMEM((NBAT, B), jnp.int32),
        pltpu.VMEM((ZPT,), jnp.float32),
        pltpu.VMEM((CPT,), jnp.float32),
    ],
)
def _counts(*args):
    _cnt_body(*args)


def _agg_body(x2d, src4, dst_hbm, z2, out, acc, src_v, dst_v,
              rows0, rows1, gsem0, gsem1, ssem0, ssem1):
    c = lax.axis_index("c")
    s = lax.axis_index("s")
    if True:
        for qi in range(2):
            q = c * 2 + qi
            pltpu.sync_copy(z2, rows0)
            for k in range(ZPT // B):
                pltpu.sync_copy(rows0, acc.at[pl.ds(s * ZPT + k * B, B)])
            plsc.subcore_barrier()

            for h in range(2):
                pltpu.sync_copy(src4.at[q, s, pl.ds(h * HALF, HALF)], src_v)
                pltpu.sync_copy(dst_hbm.at[s, pl.ds(h * HALF, HALF)], dst_v)
                pltpu.async_copy(x2d.at[src_v.at[0]], rows0, gsem0)

                @pl.loop(0, HALF, step=2)
                def _(j):
                    pltpu.async_copy(x2d.at[src_v.at[j + 1]], rows1, gsem1)
                    pltpu.make_async_copy(x2d.at[src_v.at[j]], rows0,
                                          gsem0).wait()
                    pltpu.sync_copy(rows0, acc.at[dst_v.at[j]], add=True)

                    @pl.when(j + 2 < HALF)
                    def _():
                        pltpu.async_copy(x2d.at[src_v.at[j + 2]], rows0, gsem0)

                    pltpu.make_async_copy(x2d.at[src_v.at[j + 1]], rows1,
                                          gsem1).wait()
                    pltpu.sync_copy(rows1, acc.at[dst_v.at[j + 1]], add=True)

            plsc.subcore_barrier()
            for k in range(ZPT // B):
                r0 = s * ZPT + k * B
                pltpu.sync_copy(acc.at[pl.ds(r0, B)], rows0)
                pltpu.sync_copy(rows0, out.at[q, pl.ds(r0, B)])
            plsc.subcore_barrier()


@functools.partial(
    pl.kernel, mesh=_MESH,
    out_type=jax.ShapeDtypeStruct((4, NPAD, QW), jnp.float32),
    scratch_types=[
        pltpu.VMEM_SHARED((NPAD, QW), jnp.float32),
        pltpu.VMEM((HALF, B), jnp.int32),
        pltpu.VMEM((HALF, B), jnp.int32),
        pltpu.VMEM((B, QW), jnp.float32),
        pltpu.VMEM((B, QW), jnp.float32),
        pltpu.SemaphoreType.DMA,
        pltpu.SemaphoreType.DMA,
        pltpu.SemaphoreType.DMA,
        pltpu.SemaphoreType.DMA,
    ],
)
def _aggregate(*args):
    _agg_body(*args)


def kernel(x_window, edge_index_near, edge_index_close, edge_index_sim,
           W_pre, W_post,
           Wl_0_near, bl_0_near, Wr_0_near,
           Wl_0_close, bl_0_close, Wr_0_close,
           Wl_0_sim, bl_0_sim, Wr_0_sim,
           Wl_1_near, bl_1_near, Wr_1_near,
           Wl_1_close, bl_1_close, Wr_1_close,
           Wl_1_sim, bl_1_sim, Wr_1_sim,
           Wl_2_near, bl_2_near, Wr_2_near,
           Wl_2_close, bl_2_close, Wr_2_close,
           Wl_2_sim, bl_2_sim, Wr_2_sim,
           Wl_3_near, bl_3_near, Wr_3_near,
           Wl_3_close, bl_3_close, Wr_3_close,
           Wl_3_sim, bl_3_sim, Wr_3_sim,
           W_lin, b_lin):
    params = dict(locals())
    rels = ("near", "close", "sim")

    pad = jnp.arange(EPAD - E, dtype=jnp.int32)
    pad_src = (pad * 41) % N
    pad_dst = N + (pad % (NPAD - N))
    src4s, dst3s = [], []
    for r in rels:
        ei = params["edge_index_" + r]
        src = jnp.concatenate([ei[0], pad_src])
        dst = jnp.concatenate([ei[1], pad_dst])
        src4 = (4 * src)[None, :] + jnp.arange(4, dtype=jnp.int32)[:, None]
        src4s.append(src4.reshape(4, NS, NBAT, B))
        dst3s.append(dst.reshape(NS, NBAT, B))

    zeros1 = jnp.zeros((ZPT,), jnp.float32)
    ones1 = jnp.ones((B,), jnp.float32)
    zeros2 = jnp.zeros((B, QW), jnp.float32)

    cnts = _counts(dst3s[0], dst3s[1], dst3s[2], zeros1, ones1)
    cnts = [cc[:N, None] for cc in cnts]

    x = _pre(x_window, W_pre, W_post)

    for l in range(L):
        wrs = (params["Wr_%d_near" % l] + params["Wr_%d_close" % l]
               + params["Wr_%d_sim" % l])
        bsum = (params["bl_%d_near" % l] + params["bl_%d_close" % l]
                + params["bl_%d_sim" % l])[None, :]
        x2d = x.reshape(4 * N, QW)
        t = _self(x, wrs, bsum)
        for i, r in enumerate(rels):
            sr = _aggregate(x2d, src4s[i], dst3s[i], zeros2)
            t = _acc(t, sr, cnts[i], params["Wl_%d_%s" % (l, r)],
                     final=(i == 2))
        x = t

    return _final(x, W_lin, b_lin[None, :])


# speedup vs baseline: 1.1703x; 1.0028x over previous
import functools

import jax
import jax.numpy as jnp
from jax import lax
from jax.experimental import pallas as pl
from jax.experimental.pallas import tpu as pltpu
from jax.experimental.pallas import tpu_sc as plsc

N = 10000
E = 160000
H = 512
OUT = 250
L = 4


R = 1024
G = (N + R - 1) // R


def _leaky(x):
    return jnp.where(x >= 0, x, 0.2 * x)


def _dotT(a, w):
    return jax.lax.dot_general(a, w, (((1,), (1,)), ((), ())),
                               preferred_element_type=jnp.float32)


def _pre_body(xw, wpre, wpost, o):
    h = _leaky(_dotT(xw[...], wpre[...]))
    o[...] = _leaky(_dotT(h, wpost[...]))


def _pre(xw, wpre, wpost):
    return pl.pallas_call(
        _pre_body,
        grid=(G,),
        in_specs=[
            pl.BlockSpec((R, H), lambda i: (i, 0)),
            pl.BlockSpec((H, H), lambda i: (0, 0)),
            pl.BlockSpec((H, H), lambda i: (0, 0)),
        ],
        out_specs=pl.BlockSpec((R, H), lambda i: (i, 0)),
        out_shape=jax.ShapeDtypeStruct((N, H), jnp.float32),
    )(xw, wpre, wpost)


def _cat(sq):
    a = sq[...]
    return jnp.concatenate([a[0], a[1], a[2], a[3]], axis=-1)


def _self_body(x, wrs, bsum, o):
    o[...] = _dotT(x[...], wrs[...]) + bsum[...]


def _self(x, wrs, bsum):
    blk = pl.BlockSpec((R, H), lambda i: (i, 0))
    return pl.pallas_call(
        _self_body,
        grid=(G,),
        in_specs=[blk, pl.BlockSpec((H, H), lambda i: (0, 0)),
                  pl.BlockSpec((1, H), lambda i: (0, 0))],
        out_specs=blk,
        out_shape=jax.ShapeDtypeStruct((N, H), jnp.float32),
    )(x, wrs, bsum)


def _acc_body(t, s, cnt, wl, o):
    ic = 1.0 / jnp.maximum(cnt[...], 1.0)
    o[...] = t[...] + _dotT(_cat(s) * ic, wl[...])


def _acc_final_body(t, s, cnt, wl, o):
    ic = 1.0 / jnp.maximum(cnt[...], 1.0)
    o[...] = _leaky((t[...] + _dotT(_cat(s) * ic, wl[...])) * (1.0 / 3.0))


def _acc(t, s, cnt, wl, final):
    blk = pl.BlockSpec((R, H), lambda i: (i, 0))
    sblk = pl.BlockSpec((4, R, QW), lambda i: (0, i, 0))
    return pl.pallas_call(
        _acc_final_body if final else _acc_body,
        grid=(G,),
        in_specs=[blk, sblk, pl.BlockSpec((R, 1), lambda i: (i, 0)),
                  pl.BlockSpec((H, H), lambda i: (0, 0))],
        out_specs=blk,
        out_shape=jax.ShapeDtypeStruct((N, H), jnp.float32),
    )(t, s, cnt, wl)


def _final_body(x, wlin, blin, o):
    o[...] = _dotT(x[...], wlin[...]) + blin[...]


def _final(x, wlin, blin):
    return pl.pallas_call(
        _final_body,
        grid=(G,),
        in_specs=[
            pl.BlockSpec((R, H), lambda i: (i, 0)),
            pl.BlockSpec((OUT, H), lambda i: (0, 0)),
            pl.BlockSpec((1, OUT), lambda i: (0, 0)),
        ],
        out_specs=pl.BlockSpec((R, OUT), lambda i: (i, 0)),
        out_shape=jax.ShapeDtypeStruct((N, OUT), jnp.float32),
    )(x, wlin, blin)


NC, NS = 2, 16
B = 128
NBAT = 80
HALF = NBAT // 2
EPT = NBAT * B
EPAD = EPT * NS
NPAD = 10240
ZPT = NPAD // NS
QW = 128

_MESH = plsc.VectorSubcoreMesh(core_axis_name="c", subcore_axis_name="s")


CPT = NPAD // 2 // NS


def _cnt_body(d0, d1, d2, z1, ones1, o0, o1, o2, acc, ones_v, dst_v, zb, fb):
    c = lax.axis_index("c")
    s = lax.axis_index("s")
    pltpu.sync_copy(ones1, ones_v)
    pltpu.sync_copy(z1, zb)
    for rel in range(3):
        dst_hbm = (d0, d1, d2)[rel]
        out = (o0, o1, o2)[rel]
        pltpu.sync_copy(zb, acc.at[pl.ds(s * ZPT, ZPT)])
        pltpu.sync_copy(dst_hbm.at[s], dst_v)
        plsc.subcore_barrier()

        @pl.loop(0, NBAT)
        def _(j):
            pltpu.sync_copy(ones_v, acc.at[dst_v.at[j]], add=True)

        plsc.subcore_barrier()
        off = c * (NPAD // 2) + s * CPT
        pltpu.sync_copy(acc.at[pl.ds(off, CPT)], fb)
        pltpu.sync_copy(fb, out.at[pl.ds(off, CPT)])
        plsc.subcore_barrier()


@functools.partial(
    pl.kernel, mesh=_MESH,
    out_type=[jax.ShapeDtypeStruct((NPAD,), jnp.float32)] * 3,
    scratch_types=[
        pltpu.VMEM_SHARED((NPAD,), jnp.float32),
        pltpu.VMEM((B,), jnp.float32),
        pltpu.VMEM((NBAT, B), jnp.int32),
        pltpu.VMEM((ZPT,), jnp.float32),
        pltpu.VMEM((CPT,), jnp.float32),
    ],
)
def _counts(*args):
    _cnt_body(*args)


def _agg_body(x2d, src4, dst_hbm, z2, out, acc, src_v, dst_v,
              rows0, rows1, gsem0, gsem1):
    c = lax.axis_index("c")
    s = lax.axis_index("s")
    for qi in range(2):
        q = c * 2 + qi
        pltpu.sync_copy(z2, rows0)
        for k in range(ZPT // B):
            pltpu.sync_copy(rows0, acc.at[pl.ds(s * ZPT + k * B, B)])
        plsc.subcore_barrier()

        for h in range(2):
            pltpu.sync_copy(src4.at[q, s, pl.ds(h * HALF, HALF)], src_v)
            pltpu.sync_copy(dst_hbm.at[s, pl.ds(h * HALF, HALF)], dst_v)
            pltpu.async_copy(x2d.at[src_v.at[0]], rows0, gsem0)

            @pl.loop(0, HALF, step=2)
            def _(j):
                pltpu.async_copy(x2d.at[src_v.at[j + 1]], rows1, gsem1)
                pltpu.make_async_copy(x2d.at[src_v.at[j]], rows0,
                                      gsem0).wait()
                pltpu.sync_copy(rows0, acc.at[dst_v.at[j]], add=True)

                @pl.when(j + 2 < HALF)
                def _():
                    pltpu.async_copy(x2d.at[src_v.at[j + 2]], rows0, gsem0)

                pltpu.make_async_copy(x2d.at[src_v.at[j + 1]], rows1,
                                      gsem1).wait()
                pltpu.sync_copy(rows1, acc.at[dst_v.at[j + 1]], add=True)

        plsc.subcore_barrier()
        for k in range(ZPT // B):
            r0 = s * ZPT + k * B
            pltpu.sync_copy(acc.at[pl.ds(r0, B)], rows0)
            pltpu.sync_copy(rows0, out.at[q, pl.ds(r0, B)])
        plsc.subcore_barrier()


@functools.partial(
    pl.kernel, mesh=_MESH,
    out_type=jax.ShapeDtypeStruct((4, NPAD, QW), jnp.float32),
    scratch_types=[
        pltpu.VMEM_SHARED((NPAD, QW), jnp.float32),
        pltpu.VMEM((HALF, B), jnp.int32),
        pltpu.VMEM((HALF, B), jnp.int32),
        pltpu.VMEM((B, QW), jnp.float32),
        pltpu.VMEM((B, QW), jnp.float32),
        pltpu.SemaphoreType.DMA,
        pltpu.SemaphoreType.DMA,
    ],
)
def _aggregate(*args):
    _agg_body(*args)


def kernel(x_window, edge_index_near, edge_index_close, edge_index_sim,
           W_pre, W_post,
           Wl_0_near, bl_0_near, Wr_0_near,
           Wl_0_close, bl_0_close, Wr_0_close,
           Wl_0_sim, bl_0_sim, Wr_0_sim,
           Wl_1_near, bl_1_near, Wr_1_near,
           Wl_1_close, bl_1_close, Wr_1_close,
           Wl_1_sim, bl_1_sim, Wr_1_sim,
           Wl_2_near, bl_2_near, Wr_2_near,
           Wl_2_close, bl_2_close, Wr_2_close,
           Wl_2_sim, bl_2_sim, Wr_2_sim,
           Wl_3_near, bl_3_near, Wr_3_near,
           Wl_3_close, bl_3_close, Wr_3_close,
           Wl_3_sim, bl_3_sim, Wr_3_sim,
           W_lin, b_lin):
    params = dict(locals())
    rels = ("near", "close", "sim")

    pad = jnp.arange(EPAD - E, dtype=jnp.int32)
    pad_src = (pad * 41) % N
    pad_dst = N + (pad % (NPAD - N))
    src4s, dst3s = [], []
    for r in rels:
        ei = params["edge_index_" + r]
        src = jnp.concatenate([ei[0], pad_src])
        dst = jnp.concatenate([ei[1], pad_dst])
        src4 = (4 * src)[None, :] + jnp.arange(4, dtype=jnp.int32)[:, None]
        src4s.append(src4.reshape(4, NS, NBAT, B))
        dst3s.append(dst.reshape(NS, NBAT, B))

    zeros1 = jnp.zeros((ZPT,), jnp.float32)
    ones1 = jnp.ones((B,), jnp.float32)
    zeros2 = jnp.zeros((B, QW), jnp.float32)

    cnts = _counts(dst3s[0], dst3s[1], dst3s[2], zeros1, ones1)
    cnts = [cc[:N, None] for cc in cnts]

    x = _pre(x_window, W_pre, W_post)

    for l in range(L):
        wrs = (params["Wr_%d_near" % l] + params["Wr_%d_close" % l]
               + params["Wr_%d_sim" % l])
        bsum = (params["bl_%d_near" % l] + params["bl_%d_close" % l]
                + params["bl_%d_sim" % l])[None, :]
        x2d = x.reshape(4 * N, QW)
        t = _self(x, wrs, bsum)
        for i, r in enumerate(rels):
            sr = _aggregate(x2d, src4s[i], dst3s[i], zeros2)
            t = _acc(t, sr, cnts[i], params["Wl_%d_%s" % (l, r)],
                     final=(i == 2))
        x = t

    return _final(x, W_lin, b_lin[None, :])


# speedup vs baseline: 1.1851x; 1.0126x over previous
import functools

import jax
import jax.numpy as jnp
from jax import lax
from jax.experimental import pallas as pl
from jax.experimental.pallas import tpu as pltpu
from jax.experimental.pallas import tpu_sc as plsc

N = 10000
E = 160000
H = 512
OUT = 250
L = 4


R = 1024
G = (N + R - 1) // R


def _leaky(x):
    return jnp.where(x >= 0, x, 0.2 * x)


def _dotT(a, w):
    return jax.lax.dot_general(a, w, (((1,), (1,)), ((), ())),
                               preferred_element_type=jnp.float32)


def _pre_body(xw, wpre, wpost, o):
    h = _leaky(_dotT(xw[...], wpre[...]))
    o[...] = _leaky(_dotT(h, wpost[...]))


def _pre(xw, wpre, wpost):
    return pl.pallas_call(
        _pre_body,
        grid=(G,),
        in_specs=[
            pl.BlockSpec((R, H), lambda i: (i, 0)),
            pl.BlockSpec((H, H), lambda i: (0, 0)),
            pl.BlockSpec((H, H), lambda i: (0, 0)),
        ],
        out_specs=pl.BlockSpec((R, H), lambda i: (i, 0)),
        out_shape=jax.ShapeDtypeStruct((N, H), jnp.float32),
    )(xw, wpre, wpost)


def _cat(sq):
    a = sq[...]
    return jnp.concatenate([a[0], a[1], a[2], a[3]], axis=-1)


def _self_body(x, wrs, bsum, o):
    o[...] = _dotT(x[...], wrs[...]) + bsum[...]


def _self(x, wrs, bsum):
    blk = pl.BlockSpec((R, H), lambda i: (i, 0))
    return pl.pallas_call(
        _self_body,
        grid=(G,),
        in_specs=[blk, pl.BlockSpec((H, H), lambda i: (0, 0)),
                  pl.BlockSpec((1, H), lambda i: (0, 0))],
        out_specs=blk,
        out_shape=jax.ShapeDtypeStruct((N, H), jnp.float32),
    )(x, wrs, bsum)


def _acc_body(t, s, cnt, wl, o):
    ic = 1.0 / jnp.maximum(cnt[...], 1.0)
    o[...] = t[...] + _dotT(_cat(s) * ic, wl[...])


def _acc_final_body(t, s, cnt, wl, o):
    ic = 1.0 / jnp.maximum(cnt[...], 1.0)
    o[...] = _leaky((t[...] + _dotT(_cat(s) * ic, wl[...])) * (1.0 / 3.0))


def _acc(t, s, cnt, wl, final):
    blk = pl.BlockSpec((R, H), lambda i: (i, 0))
    sblk = pl.BlockSpec((4, R, QW), lambda i: (0, i, 0))
    return pl.pallas_call(
        _acc_final_body if final else _acc_body,
        grid=(G,),
        in_specs=[blk, sblk, pl.BlockSpec((R, 1), lambda i: (i, 0)),
                  pl.BlockSpec((H, H), lambda i: (0, 0))],
        out_specs=blk,
        out_shape=jax.ShapeDtypeStruct((N, H), jnp.float32),
    )(t, s, cnt, wl)


def _final_body(x, wlin, blin, o):
    o[...] = _dotT(x[...], wlin[...]) + blin[...]


def _final(x, wlin, blin):
    return pl.pallas_call(
        _final_body,
        grid=(G,),
        in_specs=[
            pl.BlockSpec((R, H), lambda i: (i, 0)),
            pl.BlockSpec((OUT, H), lambda i: (0, 0)),
            pl.BlockSpec((1, OUT), lambda i: (0, 0)),
        ],
        out_specs=pl.BlockSpec((R, OUT), lambda i: (i, 0)),
        out_shape=jax.ShapeDtypeStruct((N, OUT), jnp.float32),
    )(x, wlin, blin)


NC, NS = 2, 16
B = 128
NBAT = 80
HALF = NBAT // 2
EPT = NBAT * B
EPAD = EPT * NS
NPAD = 10240
ZPT = NPAD // NS
QW = 128

_MESH = plsc.VectorSubcoreMesh(core_axis_name="c", subcore_axis_name="s")


CPT = NPAD // 2 // NS


def _cnt_body(d0, d1, d2, z1, ones1, o0, o1, o2, acc, ones_v, dst_v, zb, fb):
    c = lax.axis_index("c")
    s = lax.axis_index("s")
    pltpu.sync_copy(ones1, ones_v)
    pltpu.sync_copy(z1, zb)
    for rel in range(3):
        dst_hbm = (d0, d1, d2)[rel]
        out = (o0, o1, o2)[rel]
        pltpu.sync_copy(zb, acc.at[pl.ds(s * ZPT, ZPT)])
        pltpu.sync_copy(dst_hbm.at[s], dst_v)
        plsc.subcore_barrier()

        @pl.loop(0, NBAT)
        def _(j):
            pltpu.sync_copy(ones_v, acc.at[dst_v.at[j]], add=True)

        plsc.subcore_barrier()
        off = c * (NPAD // 2) + s * CPT
        pltpu.sync_copy(acc.at[pl.ds(off, CPT)], fb)
        pltpu.sync_copy(fb, out.at[pl.ds(off, CPT)])
        plsc.subcore_barrier()


@functools.partial(
    pl.kernel, mesh=_MESH,
    out_type=[jax.ShapeDtypeStruct((NPAD,), jnp.float32)] * 3,
    scratch_types=[
        pltpu.VMEM_SHARED((NPAD,), jnp.float32),
        pltpu.VMEM((B,), jnp.float32),
        pltpu.VMEM((NBAT, B), jnp.int32),
        pltpu.VMEM((ZPT,), jnp.float32),
        pltpu.VMEM((CPT,), jnp.float32),
    ],
)
def _counts(*args):
    _cnt_body(*args)


def _agg_body(x2d, src4, dst_hbm, z2, out, acc, src_v, dst_v,
              rows0, rows1, gsem0, gsem1):
    c = lax.axis_index("c")
    s = lax.axis_index("s")
    for qi in range(2):
        q = c * 2 + qi
        pltpu.sync_copy(z2, rows0)
        for k in range(ZPT // B):
            pltpu.sync_copy(rows0, acc.at[pl.ds(s * ZPT + k * B, B)])
        plsc.subcore_barrier()

        for h in range(2):
            pltpu.sync_copy(src4.at[q, s, pl.ds(h * HALF, HALF)], src_v)
            pltpu.sync_copy(dst_hbm.at[s, pl.ds(h * HALF, HALF)], dst_v)
            pltpu.async_copy(x2d.at[src_v.at[0]], rows0, gsem0)

            @pl.loop(0, HALF, step=2)
            def _(j):
                pltpu.async_copy(x2d.at[src_v.at[j + 1]], rows1, gsem1)
                pltpu.make_async_copy(x2d.at[src_v.at[j]], rows0,
                                      gsem0).wait()
                pltpu.sync_copy(rows0, acc.at[dst_v.at[j]], add=True)

                @pl.when(j + 2 < HALF)
                def _():
                    pltpu.async_copy(x2d.at[src_v.at[j + 2]], rows0, gsem0)

                pltpu.make_async_copy(x2d.at[src_v.at[j + 1]], rows1,
                                      gsem1).wait()
                pltpu.sync_copy(rows1, acc.at[dst_v.at[j + 1]], add=True)

        plsc.subcore_barrier()
        rws = (rows0, rows1)
        sms = (gsem0, gsem1)
        for k in range(ZPT // B):
            r0 = s * ZPT + k * B
            rw, sm = rws[k % 2], sms[k % 2]
            if k >= 2:
                p0 = s * ZPT + (k - 2) * B
                pltpu.make_async_copy(rw, out.at[q, pl.ds(p0, B)], sm).wait()
            pltpu.sync_copy(acc.at[pl.ds(r0, B)], rw)
            pltpu.async_copy(rw, out.at[q, pl.ds(r0, B)], sm)
        for k in (3, 4):
            r0 = s * ZPT + k * B
            pltpu.make_async_copy(rws[k % 2], out.at[q, pl.ds(r0, B)],
                                  sms[k % 2]).wait()
        plsc.subcore_barrier()


@functools.partial(
    pl.kernel, mesh=_MESH,
    out_type=jax.ShapeDtypeStruct((4, NPAD, QW), jnp.float32),
    scratch_types=[
        pltpu.VMEM_SHARED((NPAD, QW), jnp.float32),
        pltpu.VMEM((HALF, B), jnp.int32),
        pltpu.VMEM((HALF, B), jnp.int32),
        pltpu.VMEM((B, QW), jnp.float32),
        pltpu.VMEM((B, QW), jnp.float32),
        pltpu.SemaphoreType.DMA,
        pltpu.SemaphoreType.DMA,
    ],
)
def _aggregate(*args):
    _agg_body(*args)


def kernel(x_window, edge_index_near, edge_index_close, edge_index_sim,
           W_pre, W_post,
           Wl_0_near, bl_0_near, Wr_0_near,
           Wl_0_close, bl_0_close, Wr_0_close,
           Wl_0_sim, bl_0_sim, Wr_0_sim,
           Wl_1_near, bl_1_near, Wr_1_near,
           Wl_1_close, bl_1_close, Wr_1_close,
           Wl_1_sim, bl_1_sim, Wr_1_sim,
           Wl_2_near, bl_2_near, Wr_2_near,
           Wl_2_close, bl_2_close, Wr_2_close,
           Wl_2_sim, bl_2_sim, Wr_2_sim,
           Wl_3_near, bl_3_near, Wr_3_near,
           Wl_3_close, bl_3_close, Wr_3_close,
           Wl_3_sim, bl_3_sim, Wr_3_sim,
           W_lin, b_lin):
    params = dict(locals())
    rels = ("near", "close", "sim")

    pad = jnp.arange(EPAD - E, dtype=jnp.int32)
    pad_src = (pad * 41) % N
    pad_dst = N + (pad % (NPAD - N))
    src4s, dst3s = [], []
    for r in rels:
        ei = params["edge_index_" + r]
        src = jnp.concatenate([ei[0], pad_src])
        dst = jnp.concatenate([ei[1], pad_dst])
        src4 = (4 * src)[None, :] + jnp.arange(4, dtype=jnp.int32)[:, None]
        src4s.append(src4.reshape(4, NS, NBAT, B))
        dst3s.append(dst.reshape(NS, NBAT, B))

    zeros1 = jnp.zeros((ZPT,), jnp.float32)
    ones1 = jnp.ones((B,), jnp.float32)
    zeros2 = jnp.zeros((B, QW), jnp.float32)

    cnts = _counts(dst3s[0], dst3s[1], dst3s[2], zeros1, ones1)
    cnts = [cc[:N, None] for cc in cnts]

    x = _pre(x_window, W_pre, W_post)

    for l in range(L):
        wrs = (params["Wr_%d_near" % l] + params["Wr_%d_close" % l]
               + params["Wr_%d_sim" % l])
        bsum = (params["bl_%d_near" % l] + params["bl_%d_close" % l]
                + params["bl_%d_sim" % l])[None, :]
        x2d = x.reshape(4 * N, QW)
        t = _self(x, wrs, bsum)
        for i, r in enumerate(rels):
            sr = _aggregate(x2d, src4s[i], dst3s[i], zeros2)
            t = _acc(t, sr, cnts[i], params["Wl_%d_%s" % (l, r)],
                     final=(i == 2))
        x = t

    return _final(x, W_lin, b_lin[None, :])


# speedup vs baseline: 1.1871x; 1.0017x over previous
import functools

import jax
import jax.numpy as jnp
from jax import lax
from jax.experimental import pallas as pl
from jax.experimental.pallas import tpu as pltpu
from jax.experimental.pallas import tpu_sc as plsc

N = 10000
E = 160000
H = 512
OUT = 250
L = 4


R = 1024
G = (N + R - 1) // R


def _leaky(x):
    return jnp.where(x >= 0, x, 0.2 * x)


def _dotT(a, w):
    return jax.lax.dot_general(a, w, (((1,), (1,)), ((), ())),
                               preferred_element_type=jnp.float32)


def _pre_body(xw, wpre, wpost, o):
    h = _leaky(_dotT(xw[...], wpre[...]))
    o[...] = _leaky(_dotT(h, wpost[...]))


def _pre(xw, wpre, wpost):
    return pl.pallas_call(
        _pre_body,
        grid=(G,),
        in_specs=[
            pl.BlockSpec((R, H), lambda i: (i, 0)),
            pl.BlockSpec((H, H), lambda i: (0, 0)),
            pl.BlockSpec((H, H), lambda i: (0, 0)),
        ],
        out_specs=pl.BlockSpec((R, H), lambda i: (i, 0)),
        out_shape=jax.ShapeDtypeStruct((N, H), jnp.float32),
    )(xw, wpre, wpost)


def _cat(sq):
    a = sq[...]
    return jnp.concatenate([a[0], a[1], a[2], a[3]], axis=-1)


def _self_body(x, wrs, bsum, o):
    o[...] = _dotT(x[...], wrs[...]) + bsum[...]


def _self(x, wrs, bsum):
    blk = pl.BlockSpec((R, H), lambda i: (i, 0))
    return pl.pallas_call(
        _self_body,
        grid=(G,),
        in_specs=[blk, pl.BlockSpec((H, H), lambda i: (0, 0)),
                  pl.BlockSpec((1, H), lambda i: (0, 0))],
        out_specs=blk,
        out_shape=jax.ShapeDtypeStruct((N, H), jnp.float32),
    )(x, wrs, bsum)


def _acc_body(t, s, cnt, wl, o):
    ic = 1.0 / jnp.maximum(cnt[...], 1.0)
    o[...] = t[...] + _dotT(_cat(s) * ic, wl[...])


def _acc_final_body(t, s, cnt, wl, o):
    ic = 1.0 / jnp.maximum(cnt[...], 1.0)
    o[...] = _leaky((t[...] + _dotT(_cat(s) * ic, wl[...])) * (1.0 / 3.0))


def _acc(t, s, cnt, wl, final):
    blk = pl.BlockSpec((R, H), lambda i: (i, 0))
    sblk = pl.BlockSpec((4, R, QW), lambda i: (0, i, 0))
    return pl.pallas_call(
        _acc_final_body if final else _acc_body,
        grid=(G,),
        in_specs=[blk, sblk, pl.BlockSpec((R, 1), lambda i: (i, 0)),
                  pl.BlockSpec((H, H), lambda i: (0, 0))],
        out_specs=blk,
        out_shape=jax.ShapeDtypeStruct((N, H), jnp.float32),
    )(t, s, cnt, wl)


def _final_body(x, wlin, blin, o):
    o[...] = _dotT(x[...], wlin[...]) + blin[...]


def _final(x, wlin, blin):
    return pl.pallas_call(
        _final_body,
        grid=(G,),
        in_specs=[
            pl.BlockSpec((R, H), lambda i: (i, 0)),
            pl.BlockSpec((OUT, H), lambda i: (0, 0)),
            pl.BlockSpec((1, OUT), lambda i: (0, 0)),
        ],
        out_specs=pl.BlockSpec((R, OUT), lambda i: (i, 0)),
        out_shape=jax.ShapeDtypeStruct((N, OUT), jnp.float32),
    )(x, wlin, blin)


NC, NS = 2, 16
B = 128
NBAT = 80
HALF = NBAT // 2
EPT = NBAT * B
EPAD = EPT * NS
NPAD = 10240
ZPT = NPAD // NS
QW = 128

_MESH = plsc.VectorSubcoreMesh(core_axis_name="c", subcore_axis_name="s")


CPT = NPAD // 2 // NS


def _cnt_body(d0, d1, d2, z1, ones1, o0, o1, o2, acc, ones_v, dst_v, zb, fb):
    c = lax.axis_index("c")
    s = lax.axis_index("s")
    pltpu.sync_copy(ones1, ones_v)
    pltpu.sync_copy(z1, zb)
    for rel in range(3):
        dst_hbm = (d0, d1, d2)[rel]
        out = (o0, o1, o2)[rel]
        pltpu.sync_copy(zb, acc.at[pl.ds(s * ZPT, ZPT)])
        pltpu.sync_copy(dst_hbm.at[s], dst_v)
        plsc.subcore_barrier()

        @pl.loop(0, NBAT)
        def _(j):
            pltpu.sync_copy(ones_v, acc.at[dst_v.at[j]], add=True)

        plsc.subcore_barrier()
        off = c * (NPAD // 2) + s * CPT
        pltpu.sync_copy(acc.at[pl.ds(off, CPT)], fb)
        pltpu.sync_copy(fb, out.at[pl.ds(off, CPT)])
        plsc.subcore_barrier()


@functools.partial(
    pl.kernel, mesh=_MESH,
    out_type=[jax.ShapeDtypeStruct((NPAD,), jnp.float32)] * 3,
    scratch_types=[
        pltpu.VMEM_SHARED((NPAD,), jnp.float32),
        pltpu.VMEM((B,), jnp.float32),
        pltpu.VMEM((NBAT, B), jnp.int32),
        pltpu.VMEM((ZPT,), jnp.float32),
        pltpu.VMEM((CPT,), jnp.float32),
    ],
)
def _counts(*args):
    _cnt_body(*args)


def _agg_body(x2d, src4, dst_hbm, z2, out, acc, src_v, dst_v,
              rows0, rows1, gsem0, gsem1):
    c = lax.axis_index("c")
    s = lax.axis_index("s")
    for qi in range(2):
        q = c * 2 + qi
        pltpu.sync_copy(z2, rows0)
        for k in range(ZPT // B):
            pltpu.async_copy(rows0, acc.at[pl.ds(s * ZPT + k * B, B)], gsem0)
        for k in range(ZPT // B):
            pltpu.make_async_copy(rows0, acc.at[pl.ds(s * ZPT + k * B, B)],
                                  gsem0).wait()
        plsc.subcore_barrier()

        for h in range(2):
            pltpu.sync_copy(src4.at[q, s, pl.ds(h * HALF, HALF)], src_v)
            pltpu.sync_copy(dst_hbm.at[s, pl.ds(h * HALF, HALF)], dst_v)
            pltpu.async_copy(x2d.at[src_v.at[0]], rows0, gsem0)

            @pl.loop(0, HALF, step=2)
            def _(j):
                pltpu.async_copy(x2d.at[src_v.at[j + 1]], rows1, gsem1)
                pltpu.make_async_copy(x2d.at[src_v.at[j]], rows0,
                                      gsem0).wait()
                pltpu.sync_copy(rows0, acc.at[dst_v.at[j]], add=True)

                @pl.when(j + 2 < HALF)
                def _():
                    pltpu.async_copy(x2d.at[src_v.at[j + 2]], rows0, gsem0)

                pltpu.make_async_copy(x2d.at[src_v.at[j + 1]], rows1,
                                      gsem1).wait()
                pltpu.sync_copy(rows1, acc.at[dst_v.at[j + 1]], add=True)

        plsc.subcore_barrier()
        rws = (rows0, rows1)
        sms = (gsem0, gsem1)
        for k in range(ZPT // B):
            r0 = s * ZPT + k * B
            rw, sm = rws[k % 2], sms[k % 2]
            if k >= 2:
                p0 = s * ZPT + (k - 2) * B
                pltpu.make_async_copy(rw, out.at[q, pl.ds(p0, B)], sm).wait()
            pltpu.sync_copy(acc.at[pl.ds(r0, B)], rw)
            pltpu.async_copy(rw, out.at[q, pl.ds(r0, B)], sm)
        for k in (3, 4):
            r0 = s * ZPT + k * B
            pltpu.make_async_copy(rws[k % 2], out.at[q, pl.ds(r0, B)],
                                  sms[k % 2]).wait()
        plsc.subcore_barrier()


@functools.partial(
    pl.kernel, mesh=_MESH,
    out_type=jax.ShapeDtypeStruct((4, NPAD, QW), jnp.float32),
    scratch_types=[
        pltpu.VMEM_SHARED((NPAD, QW), jnp.float32),
        pltpu.VMEM((HALF, B), jnp.int32),
        pltpu.VMEM((HALF, B), jnp.int32),
        pltpu.VMEM((B, QW), jnp.float32),
        pltpu.VMEM((B, QW), jnp.float32),
        pltpu.SemaphoreType.DMA,
        pltpu.SemaphoreType.DMA,
    ],
)
def _aggregate(*args):
    _agg_body(*args)


def kernel(x_window, edge_index_near, edge_index_close, edge_index_sim,
           W_pre, W_post,
           Wl_0_near, bl_0_near, Wr_0_near,
           Wl_0_close, bl_0_close, Wr_0_close,
           Wl_0_sim, bl_0_sim, Wr_0_sim,
           Wl_1_near, bl_1_near, Wr_1_near,
           Wl_1_close, bl_1_close, Wr_1_close,
           Wl_1_sim, bl_1_sim, Wr_1_sim,
           Wl_2_near, bl_2_near, Wr_2_near,
           Wl_2_close, bl_2_close, Wr_2_close,
           Wl_2_sim, bl_2_sim, Wr_2_sim,
           Wl_3_near, bl_3_near, Wr_3_near,
           Wl_3_close, bl_3_close, Wr_3_close,
           Wl_3_sim, bl_3_sim, Wr_3_sim,
           W_lin, b_lin):
    params = dict(locals())
    rels = ("near", "close", "sim")

    pad = jnp.arange(EPAD - E, dtype=jnp.int32)
    pad_src = (pad * 41) % N
    pad_dst = N + (pad % (NPAD - N))
    src4s, dst3s = [], []
    for r in rels:
        ei = params["edge_index_" + r]
        src = jnp.concatenate([ei[0], pad_src])
        dst = jnp.concatenate([ei[1], pad_dst])
        src4 = (4 * src)[None, :] + jnp.arange(4, dtype=jnp.int32)[:, None]
        src4s.append(src4.reshape(4, NS, NBAT, B))
        dst3s.append(dst.reshape(NS, NBAT, B))

    zeros1 = jnp.zeros((ZPT,), jnp.float32)
    ones1 = jnp.ones((B,), jnp.float32)
    zeros2 = jnp.zeros((B, QW), jnp.float32)

    cnts = _counts(dst3s[0], dst3s[1], dst3s[2], zeros1, ones1)
    cnts = [cc[:N, None] for cc in cnts]

    x = _pre(x_window, W_pre, W_post)

    for l in range(L):
        wrs = (params["Wr_%d_near" % l] + params["Wr_%d_close" % l]
               + params["Wr_%d_sim" % l])
        bsum = (params["bl_%d_near" % l] + params["bl_%d_close" % l]
                + params["bl_%d_sim" % l])[None, :]
        x2d = x.reshape(4 * N, QW)
        t = _self(x, wrs, bsum)
        for i, r in enumerate(rels):
            sr = _aggregate(x2d, src4s[i], dst3s[i], zeros2)
            t = _acc(t, sr, cnts[i], params["Wl_%d_%s" % (l, r)],
                     final=(i == 2))
        x = t

    return _final(x, W_lin, b_lin[None, :])
